# Initial kernel scaffold; baseline (speedup 1.0000x reference)
import functools

import jax
import jax.numpy as jnp
from jax import lax
from jax.experimental import pallas as pl
from jax.experimental.pallas import tpu as pltpu
from jax.experimental.pallas import tpu_sc as plsc

N_NODES = 10000
N_EDGES = 320000
D = 128

NC = 2
NS = 16
NW = NC * NS
EPT = N_EDGES // NW
CHUNK = 100
NCH = EPT // CHUNK
RPS = N_NODES // NS

_MESH = plsc.VectorSubcoreMesh(
    core_axis_name="c", subcore_axis_name="s", num_cores=NC, num_subcores=NS
)


def _worker_id():
    return lax.axis_index("c") * NS + lax.axis_index("s")


@functools.partial(
    pl.kernel,
    out_type=jax.ShapeDtypeStruct((NC, N_NODES, 16), jnp.float32),
    mesh=_MESH,
    scratch_types=[
        pltpu.VMEM((NCH, CHUNK), jnp.int32),
        pltpu.VMEM((CHUNK, 16), jnp.float32),
        pltpu.VMEM((RPS, 16), jnp.float32),
        pltpu.VMEM_SHARED((N_NODES, 16), jnp.float32),
    ],
)
def _sc_deg(dst_hbm, out_hbm, dstv, onesv, zv, acc):
    c = lax.axis_index("c")
    s = lax.axis_index("s")
    wid = _worker_id()
    pltpu.sync_copy(dst_hbm.at[wid], dstv)

    ones16 = jnp.ones((16,), jnp.float32)
    zero16 = jnp.zeros((16,), jnp.float32)

    def fill_ones(i, carry):
        onesv[i, :] = ones16
        return carry

    lax.fori_loop(0, CHUNK, fill_ones, 0)

    def fill_zero(i, carry):
        zv[i, :] = zero16
        return carry

    lax.fori_loop(0, RPS, fill_zero, 0)

    pltpu.sync_copy(zv, acc.at[pl.ds(s * RPS, RPS)])
    plsc.subcore_barrier()

    def body(j, carry):
        pltpu.sync_copy(onesv, acc.at[dstv.at[j]], add=True)
        return carry

    lax.fori_loop(0, NCH, body, 0)
    plsc.subcore_barrier()
    pltpu.sync_copy(acc.at[pl.ds(s * RPS, RPS)], out_hbm.at[c, pl.ds(s * RPS, RPS)])


@functools.partial(
    pl.kernel,
    out_type=jax.ShapeDtypeStruct((NC, N_NODES, D), jnp.float32),
    mesh=_MESH,
    scratch_types=[
        pltpu.VMEM((NCH, CHUNK), jnp.int32),
        pltpu.VMEM((NCH, CHUNK), jnp.int32),
        pltpu.VMEM((CHUNK, D), jnp.float32),
        pltpu.VMEM((CHUNK, D), jnp.float32),
        pltpu.VMEM_SHARED((N_NODES, D), jnp.float32),
        pltpu.SemaphoreType.DMA,
        pltpu.SemaphoreType.DMA,
    ],
)
def _sc_agg(y_hbm, src_hbm, dst_hbm, out_hbm, srcv, dstv, rows0, rows1, acc, sem0, sem1):
    c = lax.axis_index("c")
    s = lax.axis_index("s")
    wid = _worker_id()
    pltpu.sync_copy(src_hbm.at[wid], srcv)
    pltpu.sync_copy(dst_hbm.at[wid], dstv)

    zero16 = jnp.zeros((16,), jnp.float32)

    def fill_zero(i, carry):
        for k in range(D // 16):
            rows0[i, pl.ds(k * 16, 16)] = zero16
        return carry

    lax.fori_loop(0, CHUNK, fill_zero, 0)
    for r in range(RPS // CHUNK):
        pltpu.sync_copy(rows0, acc.at[pl.ds(s * RPS + r * CHUNK, CHUNK)])
    rem = RPS % CHUNK
    if rem:
        pltpu.sync_copy(
            rows0.at[pl.ds(0, rem)],
            acc.at[pl.ds(s * RPS + (RPS // CHUNK) * CHUNK, rem)],
        )
    plsc.subcore_barrier()

    bufs = (rows0, rows1)
    sems = (sem0, sem1)
    pltpu.async_copy(y_hbm.at[srcv.at[0]], rows0, sem0)
    pltpu.async_copy(y_hbm.at[srcv.at[1]], rows1, sem1)

    def body(j, carry):
        for b in range(2):
            cidx = 2 * j + b
            pltpu.make_async_copy(y_hbm.at[srcv.at[cidx]], bufs[b], sems[b]).wait()
            pltpu.sync_copy(bufs[b], acc.at[dstv.at[cidx]], add=True)

            @pl.when(j < NCH // 2 - 1)
            def _start_next():
                pltpu.async_copy(y_hbm.at[srcv.at[cidx + 2]], bufs[b], sems[b])

        return carry

    lax.fori_loop(0, NCH // 2, body, 0)
    plsc.subcore_barrier()
    pltpu.sync_copy(acc.at[pl.ds(s * RPS, RPS)], out_hbm.at[c, pl.ds(s * RPS, RPS)])


BLK = 2000


def _dis_from(dp):
    deg = dp[0, :, 0:1] + dp[1, :, 0:1] + 1.0
    return lax.rsqrt(deg)


def _tc_first_body(x_ref, w_ref, dp_ref, y_ref):
    dis = _dis_from(dp_ref[...])
    xw = jnp.dot(x_ref[...], w_ref[...], preferred_element_type=jnp.float32)
    y_ref[...] = dis * xw


def _tc_mid_body(agg_ref, y1_ref, dp_ref, b_ref, w_ref, y2_ref):
    dis = _dis_from(dp_ref[...])
    a = agg_ref[...]
    h = jnp.maximum(dis * (a[0] + a[1] + y1_ref[...]) + b_ref[...], 0.0)
    y2_ref[...] = dis * jnp.dot(h, w_ref[...], preferred_element_type=jnp.float32)


def _tc_last_body(agg_ref, y2_ref, dp_ref, b_ref, out_ref):
    dis = _dis_from(dp_ref[...])
    a = agg_ref[...]
    out_ref[...] = dis * (a[0] + a[1] + y2_ref[...]) + b_ref[...]


def _row_spec(width):
    return pl.BlockSpec((BLK, width), lambda i: (i, 0))


def _pair_spec(width):
    return pl.BlockSpec((2, BLK, width), lambda i: (0, i, 0))


def _full_spec(shape):
    return pl.BlockSpec(shape, lambda i: tuple(0 for _ in shape))


def _tc_first(x, w, dp):
    return pl.pallas_call(
        _tc_first_body,
        grid=(N_NODES // BLK,),
        in_specs=[_row_spec(D), _full_spec((D, D)), _pair_spec(16)],
        out_specs=_row_spec(D),
        out_shape=jax.ShapeDtypeStruct((N_NODES, D), jnp.float32),
    )(x, w, dp)


def _tc_mid(agg, y1, dp, b, w):
    return pl.pallas_call(
        _tc_mid_body,
        grid=(N_NODES // BLK,),
        in_specs=[
            _pair_spec(D),
            _row_spec(D),
            _pair_spec(16),
            _full_spec((1, D)),
            _full_spec((D, D)),
        ],
        out_specs=_row_spec(D),
        out_shape=jax.ShapeDtypeStruct((N_NODES, D), jnp.float32),
    )(agg, y1, dp, b, w)


def _tc_last(agg, y2, dp, b):
    return pl.pallas_call(
        _tc_last_body,
        grid=(N_NODES // BLK,),
        in_specs=[_pair_spec(D), _row_spec(D), _pair_spec(16), _full_spec((1, D))],
        out_specs=_row_spec(D),
        out_shape=jax.ShapeDtypeStruct((N_NODES, D), jnp.float32),
    )(agg, y2, dp, b)


def kernel(x, edge_index, W1, b1, W2, b2):
    src = edge_index[0].astype(jnp.int32).reshape(NW, NCH, CHUNK)
    dst = edge_index[1].astype(jnp.int32).reshape(NW, NCH, CHUNK)
    b1r = b1.reshape(1, D)
    b2r = b2.reshape(1, D)

    dp = _sc_deg(dst)
    y1 = _tc_first(x, W1, dp)
    agg1 = _sc_agg(y1, src, dst)
    y2 = _tc_mid(agg1, y1, dp, b1r, W2)
    agg2 = _sc_agg(y2, src, dst)
    return _tc_last(agg2, y2, dp, b2r)


# speedup vs baseline: 31.8314x; 31.8314x over previous
import functools

import jax
import jax.numpy as jnp
from jax import lax
from jax.experimental import pallas as pl
from jax.experimental.pallas import tpu as pltpu
from jax.experimental.pallas import tpu_sc as plsc

N_NODES = 10000
N_EDGES = 320000
D = 128

NC = 2
NS = 16
NW = NC * NS
EPT = N_EDGES // NW
CHUNK = 80
NCH = EPT // CHUNK
NPAD = 10240
RPS = NPAD // NS

_MESH = plsc.VectorSubcoreMesh(
    core_axis_name="c", subcore_axis_name="s", num_cores=NC, num_subcores=NS
)
_SC_PARAMS = pltpu.CompilerParams(needs_layout_passes=False)

def _z16():
    return jnp.zeros((16,), jnp.float32)


@functools.partial(
    pl.kernel,
    out_type=jax.ShapeDtypeStruct((NC, NPAD), jnp.float32),
    mesh=_MESH,
    compiler_params=_SC_PARAMS,
    scratch_types=[
        pltpu.VMEM((EPT,), jnp.int32),
        pltpu.VMEM((NPAD,), jnp.float32),
        pltpu.VMEM((NS, RPS), jnp.float32),
        pltpu.VMEM((RPS,), jnp.float32),
        pltpu.VMEM_SHARED((NS, NPAD), jnp.float32),
    ],
)
def _sc_deg(dst_hbm, out_hbm, dstv, hist, stage, degl, shared):
    c = lax.axis_index("c")
    s = lax.axis_index("s")
    wid = c * NS + s
    pltpu.sync_copy(dst_hbm.at[wid], dstv)

    def zero_hist(i, carry):
        hist[pl.ds(i * 16, 16)] = _z16()
        return carry

    lax.fori_loop(0, NPAD // 16, zero_hist, 0)

    def count(i, carry):
        idx = dstv[pl.ds(i * 16, 16)]
        plsc.addupdate_scatter(hist, [idx], _z16() + 1.0)
        return carry

    lax.fori_loop(0, EPT // 16, count, 0)

    pltpu.sync_copy(hist, shared.at[s])
    plsc.subcore_barrier()
    pltpu.sync_copy(shared.at[:, pl.ds(s * RPS, RPS)], stage)

    def reduce_cols(k, carry):
        acc = stage[0, pl.ds(k * 16, 16)]
        for r in range(1, NS):
            acc = acc + stage[r, pl.ds(k * 16, 16)]
        degl[pl.ds(k * 16, 16)] = acc
        return carry

    lax.fori_loop(0, RPS // 16, reduce_cols, 0)
    pltpu.sync_copy(degl, out_hbm.at[c, pl.ds(s * RPS, RPS)])


@functools.partial(
    pl.kernel,
    out_type=jax.ShapeDtypeStruct((NC, NPAD, D), jnp.float32),
    mesh=_MESH,
    compiler_params=_SC_PARAMS,
    scratch_types=[
        pltpu.VMEM((NCH, CHUNK), jnp.int32),
        pltpu.VMEM((CHUNK,), jnp.int32),
        pltpu.VMEM((CHUNK,), jnp.int32),
        pltpu.VMEM((CHUNK,), jnp.int32),
        pltpu.VMEM((CHUNK, D), jnp.float32),
        pltpu.VMEM((CHUNK, D), jnp.float32),
        pltpu.VMEM_SHARED((NPAD, D), jnp.float32),
        pltpu.SemaphoreType.DMA,
        pltpu.SemaphoreType.DMA,
    ],
)
def _sc_agg(y_hbm, pk_hbm, out_hbm, pkv, srcb0, srcb1, dstb, rows0, rows1, acc, sem0, sem1):
    c = lax.axis_index("c")
    s = lax.axis_index("s")
    wid = c * NS + s
    pltpu.sync_copy(pk_hbm.at[wid], pkv)

    def unpack_src(cidx, dst_ref):
        for k in range(CHUNK // 16):
            p = pkv[cidx, pl.ds(k * 16, 16)]
            dst_ref[pl.ds(k * 16, 16)] = lax.bitwise_and(p, 0xFFFF)

    def unpack_dst(cidx):
        for k in range(CHUNK // 16):
            p = pkv[cidx, pl.ds(k * 16, 16)]
            dstb[pl.ds(k * 16, 16)] = lax.shift_right_logical(p, 16)

    def fill_zero(i, carry):
        for k in range(D // 16):
            rows0[i, pl.ds(k * 16, 16)] = _z16()
        return carry

    lax.fori_loop(0, CHUNK, fill_zero, 0)
    for r in range(RPS // CHUNK):
        pltpu.sync_copy(rows0, acc.at[pl.ds(s * RPS + r * CHUNK, CHUNK)])
    plsc.subcore_barrier()

    bufs = (rows0, rows1)
    sems = (sem0, sem1)
    srcbs = (srcb0, srcb1)
    unpack_src(0, srcb0)
    pltpu.async_copy(y_hbm.at[srcb0], rows0, sem0)
    unpack_src(1, srcb1)
    pltpu.async_copy(y_hbm.at[srcb1], rows1, sem1)

    def body(j, carry):
        for b in range(2):
            cidx = 2 * j + b
            pltpu.make_async_copy(y_hbm.at[srcbs[b]], bufs[b], sems[b]).wait()
            unpack_dst(cidx)
            pltpu.sync_copy(bufs[b], acc.at[dstb], add=True)

            @pl.when(j < (NCH - 1) // 2 - 1)
            def _start_next():
                unpack_src(cidx + 2, srcbs[b])
                pltpu.async_copy(y_hbm.at[srcbs[b]], bufs[b], sems[b])

        return carry

    lax.fori_loop(0, (NCH - 1) // 2, body, 0)
    unpack_src(NCH - 1, srcb0)
    pltpu.async_copy(y_hbm.at[srcb0], rows0, sem0).wait()
    unpack_dst(NCH - 1)
    pltpu.sync_copy(rows0, acc.at[dstb], add=True)

    plsc.subcore_barrier()
    pltpu.sync_copy(acc.at[pl.ds(s * RPS, RPS)], out_hbm.at[c, pl.ds(s * RPS, RPS)])


BLK = 2000


def _dis_from(dt):
    deg = dt[:, 0:1] + dt[:, 1:2] + 1.0
    return lax.rsqrt(deg)


def _tc_first_body(x_ref, w_ref, dt_ref, y_ref):
    dis = _dis_from(dt_ref[...])
    xw = jnp.dot(x_ref[...], w_ref[...], preferred_element_type=jnp.float32)
    y_ref[...] = dis * xw


def _tc_mid_body(agg_ref, y1_ref, dt_ref, b_ref, w_ref, y2_ref):
    dis = _dis_from(dt_ref[...])
    a = agg_ref[...]
    h = jnp.maximum(dis * (a[0] + a[1] + y1_ref[...]) + b_ref[...], 0.0)
    y2_ref[...] = dis * jnp.dot(h, w_ref[...], preferred_element_type=jnp.float32)


def _tc_last_body(agg_ref, y2_ref, dt_ref, b_ref, out_ref):
    dis = _dis_from(dt_ref[...])
    a = agg_ref[...]
    out_ref[...] = dis * (a[0] + a[1] + y2_ref[...]) + b_ref[...]


def _row_spec(width):
    return pl.BlockSpec((BLK, width), lambda i: (i, 0))


def _pair_spec(width):
    return pl.BlockSpec((2, BLK, width), lambda i: (0, i, 0))


def _full_spec(shape):
    return pl.BlockSpec(shape, lambda i: tuple(0 for _ in shape))


def _tc_first(x, w, dt):
    return pl.pallas_call(
        _tc_first_body,
        grid=(N_NODES // BLK,),
        in_specs=[_row_spec(D), _full_spec((D, D)), _row_spec(2)],
        out_specs=_row_spec(D),
        out_shape=jax.ShapeDtypeStruct((N_NODES, D), jnp.float32),
    )(x, w, dt)


def _tc_mid(agg, y1, dt, b, w):
    return pl.pallas_call(
        _tc_mid_body,
        grid=(N_NODES // BLK,),
        in_specs=[
            _pair_spec(D),
            _row_spec(D),
            _row_spec(2),
            _full_spec((1, D)),
            _full_spec((D, D)),
        ],
        out_specs=_row_spec(D),
        out_shape=jax.ShapeDtypeStruct((N_NODES, D), jnp.float32),
    )(agg, y1, dt, b, w)


def _tc_last(agg, y2, dt, b):
    return pl.pallas_call(
        _tc_last_body,
        grid=(N_NODES // BLK,),
        in_specs=[_pair_spec(D), _row_spec(D), _row_spec(2), _full_spec((1, D))],
        out_specs=_row_spec(D),
        out_shape=jax.ShapeDtypeStruct((N_NODES, D), jnp.float32),
    )(agg, y2, dt, b)


def kernel(x, edge_index, W1, b1, W2, b2):
    src = edge_index[0].astype(jnp.int32)
    dst = edge_index[1].astype(jnp.int32)
    dst_flat = dst.reshape(NW, EPT)
    packed = (src | (dst << 16)).reshape(NW, NCH, CHUNK)
    b1r = b1.reshape(1, D)
    b2r = b2.reshape(1, D)

    dp = _sc_deg(dst_flat)
    dt = dp.T
    y1 = _tc_first(x, W1, dt)
    agg1 = _sc_agg(y1, packed)
    y2 = _tc_mid(agg1, y1, dt, b1r, W2)
    agg2 = _sc_agg(y2, packed)
    return _tc_last(agg2, y2, dt, b2r)


# speedup vs baseline: 31.9708x; 1.0044x over previous
import functools

import jax
import jax.numpy as jnp
from jax import lax
from jax.experimental import pallas as pl
from jax.experimental.pallas import tpu as pltpu
from jax.experimental.pallas import tpu_sc as plsc

N_NODES = 10000
N_EDGES = 320000
D = 128

NC = 2
NS = 16
NW = NC * NS
EPT = N_EDGES // NW
CHUNK = 80
NCH = EPT // CHUNK
NPAD = 10240
RPS = NPAD // NS

_MESH = plsc.VectorSubcoreMesh(
    core_axis_name="c", subcore_axis_name="s", num_cores=NC, num_subcores=NS
)
_SC_PARAMS = pltpu.CompilerParams(needs_layout_passes=False)

def _z16():
    return jnp.zeros((16,), jnp.float32)


@functools.partial(
    pl.kernel,
    out_type=jax.ShapeDtypeStruct((NC, NPAD), jnp.float32),
    mesh=_MESH,
    compiler_params=_SC_PARAMS,
    scratch_types=[
        pltpu.VMEM((EPT,), jnp.int32),
        pltpu.VMEM((NPAD,), jnp.float32),
        pltpu.VMEM((NS, RPS), jnp.float32),
        pltpu.VMEM((RPS,), jnp.float32),
        pltpu.VMEM_SHARED((NS, NPAD), jnp.float32),
    ],
)
def _sc_deg(dst_hbm, out_hbm, dstv, hist, stage, degl, shared):
    c = lax.axis_index("c")
    s = lax.axis_index("s")
    wid = c * NS + s
    pltpu.sync_copy(dst_hbm.at[wid], dstv)

    def zero_hist(i, carry):
        hist[pl.ds(i * 16, 16)] = _z16()
        return carry

    lax.fori_loop(0, NPAD // 16, zero_hist, 0)

    def count(i, carry):
        idx = dstv[pl.ds(i * 16, 16)]
        plsc.addupdate_scatter(hist, [idx], _z16() + 1.0)
        return carry

    lax.fori_loop(0, EPT // 16, count, 0)

    pltpu.sync_copy(hist, shared.at[s])
    plsc.subcore_barrier()
    pltpu.sync_copy(shared.at[:, pl.ds(s * RPS, RPS)], stage)

    def reduce_cols(k, carry):
        acc = stage[0, pl.ds(k * 16, 16)]
        for r in range(1, NS):
            acc = acc + stage[r, pl.ds(k * 16, 16)]
        degl[pl.ds(k * 16, 16)] = acc
        return carry

    lax.fori_loop(0, RPS // 16, reduce_cols, 0)
    pltpu.sync_copy(degl, out_hbm.at[c, pl.ds(s * RPS, RPS)])


@functools.partial(
    pl.kernel,
    out_type=jax.ShapeDtypeStruct((NC, NPAD, D), jnp.float32),
    mesh=_MESH,
    compiler_params=_SC_PARAMS,
    scratch_types=[
        pltpu.VMEM((EPT,), jnp.int32),
        pltpu.VMEM((2, CHUNK), jnp.int32),
        pltpu.VMEM((1, CHUNK), jnp.int32),
        [pltpu.VMEM((CHUNK, D), jnp.float32) for _ in range(2)],
        pltpu.VMEM_SHARED((NPAD, D), jnp.float32),
        [pltpu.SemaphoreType.DMA for _ in range(2)],
    ],
)
def _sc_agg(y_hbm, pk_hbm, out_hbm, pkv, srcb2, dstb2, bufs, acc, gsems):
    c = lax.axis_index("c")
    s = lax.axis_index("s")
    wid = c * NS + s
    pltpu.sync_copy(pk_hbm.at[wid], pkv)

    def unpack_src(cidx, b):
        for k in range(CHUNK // 16):
            p = pkv[pl.ds(cidx * CHUNK + k * 16, 16)]
            srcb2[b, pl.ds(k * 16, 16)] = lax.bitwise_and(p, 0xFFFF)

    def unpack_dst(cidx):
        for k in range(CHUNK // 16):
            p = pkv[pl.ds(cidx * CHUNK + k * 16, 16)]
            dstb2[0, pl.ds(k * 16, 16)] = lax.shift_right_logical(p, 16)

    def fill_zero(i, carry):
        for k in range(D // 16):
            bufs[0][i, pl.ds(k * 16, 16)] = _z16()
        return carry

    lax.fori_loop(0, CHUNK, fill_zero, 0)
    for r in range(RPS // 80):
        pltpu.sync_copy(bufs[0].at[pl.ds(0, 80)],
                        acc.at[pl.ds(s * RPS + r * 80, 80)])
    plsc.subcore_barrier()

    def wait_gather(b):
        pltpu.make_async_copy(y_hbm.at[srcb2.at[b]], bufs[b], gsems[b]).wait()

    unpack_src(0, 0)
    pltpu.async_copy(y_hbm.at[srcb2.at[0]], bufs[0], gsems[0])
    unpack_src(1, 1)
    pltpu.async_copy(y_hbm.at[srcb2.at[1]], bufs[1], gsems[1])

    def body(j, carry):
        for b in range(2):
            cidx = 2 * j + b
            wait_gather(b)
            unpack_dst(cidx)
            pltpu.sync_copy(bufs[b], acc.at[dstb2.at[0]], add=True)

            @pl.when(j < (NCH - 1) // 2 - 1)
            def _start_next():
                unpack_src(cidx + 2, b)
                pltpu.async_copy(y_hbm.at[srcb2.at[b]], bufs[b], gsems[b])

        return carry

    lax.fori_loop(0, (NCH - 1) // 2, body, 0)
    unpack_src(NCH - 1, 0)
    pltpu.async_copy(y_hbm.at[srcb2.at[0]], bufs[0], gsems[0]).wait()
    unpack_dst(NCH - 1)
    pltpu.sync_copy(bufs[0], acc.at[dstb2.at[0]], add=True)

    plsc.subcore_barrier()
    pltpu.sync_copy(acc.at[pl.ds(s * RPS, RPS)], out_hbm.at[c, pl.ds(s * RPS, RPS)])


BLK = 2000


def _dis_from(dt):
    deg = dt[:, 0:1] + dt[:, 1:2] + 1.0
    return lax.rsqrt(deg)


def _tc_first_body(x_ref, w_ref, dt_ref, y_ref):
    dis = _dis_from(dt_ref[...])
    xw = jnp.dot(x_ref[...], w_ref[...], preferred_element_type=jnp.float32)
    y_ref[...] = dis * xw


def _tc_mid_body(agg_ref, y1_ref, dt_ref, b_ref, w_ref, y2_ref):
    dis = _dis_from(dt_ref[...])
    a = agg_ref[...]
    h = jnp.maximum(dis * (a[0] + a[1] + y1_ref[...]) + b_ref[...], 0.0)
    y2_ref[...] = dis * jnp.dot(h, w_ref[...], preferred_element_type=jnp.float32)


def _tc_last_body(agg_ref, y2_ref, dt_ref, b_ref, out_ref):
    dis = _dis_from(dt_ref[...])
    a = agg_ref[...]
    out_ref[...] = dis * (a[0] + a[1] + y2_ref[...]) + b_ref[...]


def _row_spec(width):
    return pl.BlockSpec((BLK, width), lambda i: (i, 0))


def _pair_spec(width):
    return pl.BlockSpec((2, BLK, width), lambda i: (0, i, 0))


def _full_spec(shape):
    return pl.BlockSpec(shape, lambda i: tuple(0 for _ in shape))


def _tc_first(x, w, dt):
    return pl.pallas_call(
        _tc_first_body,
        grid=(N_NODES // BLK,),
        in_specs=[_row_spec(D), _full_spec((D, D)), _row_spec(2)],
        out_specs=_row_spec(D),
        out_shape=jax.ShapeDtypeStruct((N_NODES, D), jnp.float32),
    )(x, w, dt)


def _tc_mid(agg, y1, dt, b, w):
    return pl.pallas_call(
        _tc_mid_body,
        grid=(N_NODES // BLK,),
        in_specs=[
            _pair_spec(D),
            _row_spec(D),
            _row_spec(2),
            _full_spec((1, D)),
            _full_spec((D, D)),
        ],
        out_specs=_row_spec(D),
        out_shape=jax.ShapeDtypeStruct((N_NODES, D), jnp.float32),
    )(agg, y1, dt, b, w)


def _tc_last(agg, y2, dt, b):
    return pl.pallas_call(
        _tc_last_body,
        grid=(N_NODES // BLK,),
        in_specs=[_pair_spec(D), _row_spec(D), _row_spec(2), _full_spec((1, D))],
        out_specs=_row_spec(D),
        out_shape=jax.ShapeDtypeStruct((N_NODES, D), jnp.float32),
    )(agg, y2, dt, b)


def kernel(x, edge_index, W1, b1, W2, b2):
    src = edge_index[0].astype(jnp.int32)
    dst = edge_index[1].astype(jnp.int32)
    dst_flat = dst.reshape(NW, EPT)
    packed = (src | (dst << 16)).reshape(NW, EPT)
    b1r = b1.reshape(1, D)
    b2r = b2.reshape(1, D)

    dp = _sc_deg(dst_flat)
    dt = dp.T
    y1 = _tc_first(x, W1, dt)
    agg1 = _sc_agg(y1, packed)
    y2 = _tc_mid(agg1, y1, dt, b1r, W2)
    agg2 = _sc_agg(y2, packed)
    return _tc_last(agg2, y2, dt, b2r)


# speedup vs baseline: 34.1231x; 1.0673x over previous
import functools

import jax
import jax.numpy as jnp
from jax import lax
from jax.experimental import pallas as pl
from jax.experimental.pallas import tpu as pltpu
from jax.experimental.pallas import tpu_sc as plsc

N_NODES = 10000
N_EDGES = 320000
D = 128

NC = 2
NS = 16
NW = NC * NS
EPT = N_EDGES // NW
CHUNK = 112
NCH = 89
TAIL = EPT - NCH * CHUNK
NPAD = 10240
RPS = NPAD // NS

_MESH = plsc.VectorSubcoreMesh(
    core_axis_name="c", subcore_axis_name="s", num_cores=NC, num_subcores=NS
)
_SC_PARAMS = pltpu.CompilerParams(needs_layout_passes=False)

def _z16():
    return jnp.zeros((16,), jnp.float32)


@functools.partial(
    pl.kernel,
    out_type=jax.ShapeDtypeStruct((NC, NPAD), jnp.float32),
    mesh=_MESH,
    compiler_params=_SC_PARAMS,
    scratch_types=[
        pltpu.VMEM((EPT,), jnp.int32),
        pltpu.VMEM((NPAD,), jnp.float32),
        pltpu.VMEM((NS, RPS), jnp.float32),
        pltpu.VMEM((RPS,), jnp.float32),
        pltpu.VMEM_SHARED((NS, NPAD), jnp.float32),
    ],
)
def _sc_deg(dst_hbm, out_hbm, dstv, hist, stage, degl, shared):
    c = lax.axis_index("c")
    s = lax.axis_index("s")
    wid = c * NS + s
    pltpu.sync_copy(dst_hbm.at[wid], dstv)

    def zero_hist(i, carry):
        hist[pl.ds(i * 16, 16)] = _z16()
        return carry

    lax.fori_loop(0, NPAD // 16, zero_hist, 0)

    def count(i, carry):
        idx = dstv[pl.ds(i * 16, 16)]
        plsc.addupdate_scatter(hist, [idx], _z16() + 1.0)
        return carry

    lax.fori_loop(0, EPT // 16, count, 0)

    pltpu.sync_copy(hist, shared.at[s])
    plsc.subcore_barrier()
    pltpu.sync_copy(shared.at[:, pl.ds(s * RPS, RPS)], stage)

    def reduce_cols(k, carry):
        acc = stage[0, pl.ds(k * 16, 16)]
        for r in range(1, NS):
            acc = acc + stage[r, pl.ds(k * 16, 16)]
        degl[pl.ds(k * 16, 16)] = acc
        return carry

    lax.fori_loop(0, RPS // 16, reduce_cols, 0)
    pltpu.sync_copy(degl, out_hbm.at[c, pl.ds(s * RPS, RPS)])


@functools.partial(
    pl.kernel,
    out_type=jax.ShapeDtypeStruct((NC, NPAD, D), jnp.float32),
    mesh=_MESH,
    compiler_params=_SC_PARAMS,
    scratch_types=[
        pltpu.VMEM((EPT,), jnp.int32),
        pltpu.VMEM((2, CHUNK), jnp.int32),
        pltpu.VMEM((1, CHUNK), jnp.int32),
        pltpu.VMEM((1, TAIL), jnp.int32),
        pltpu.VMEM((1, TAIL), jnp.int32),
        [pltpu.VMEM((CHUNK, D), jnp.float32) for _ in range(2)],
        pltpu.VMEM_SHARED((NPAD, D), jnp.float32),
        [pltpu.SemaphoreType.DMA for _ in range(2)],
    ],
)
def _sc_agg(y_hbm, pk_hbm, out_hbm, pkv, srcb2, dstb2, srcbt, dstbt, bufs, acc, gsems):
    c = lax.axis_index("c")
    s = lax.axis_index("s")
    wid = c * NS + s
    pltpu.sync_copy(pk_hbm.at[wid], pkv)

    def unpack(base, n, ref, shift):
        for k in range(n // 16):
            p = pkv[pl.ds(base + k * 16, 16)]
            if shift:
                ref[0, pl.ds(k * 16, 16)] = lax.shift_right_logical(p, 16)
            else:
                ref[0, pl.ds(k * 16, 16)] = lax.bitwise_and(p, 0xFFFF)

    def unpack_src(cidx, b):
        for k in range(CHUNK // 16):
            p = pkv[pl.ds(cidx * CHUNK + k * 16, 16)]
            srcb2[b, pl.ds(k * 16, 16)] = lax.bitwise_and(p, 0xFFFF)

    def unpack_dst(cidx):
        for k in range(CHUNK // 16):
            p = pkv[pl.ds(cidx * CHUNK + k * 16, 16)]
            dstb2[0, pl.ds(k * 16, 16)] = lax.shift_right_logical(p, 16)

    def fill_zero(i, carry):
        for k in range(D // 16):
            bufs[0][i, pl.ds(k * 16, 16)] = _z16()
        return carry

    lax.fori_loop(0, 80, fill_zero, 0)
    for r in range(RPS // 80):
        pltpu.sync_copy(bufs[0].at[pl.ds(0, 80)],
                        acc.at[pl.ds(s * RPS + r * 80, 80)])
    plsc.subcore_barrier()

    def wait_gather(b):
        pltpu.make_async_copy(y_hbm.at[srcb2.at[b]], bufs[b], gsems[b]).wait()

    unpack_src(0, 0)
    pltpu.async_copy(y_hbm.at[srcb2.at[0]], bufs[0], gsems[0])
    unpack_src(1, 1)
    pltpu.async_copy(y_hbm.at[srcb2.at[1]], bufs[1], gsems[1])

    def body(j, carry):
        for b in range(2):
            cidx = 2 * j + b
            wait_gather(b)
            unpack_dst(cidx)
            pltpu.sync_copy(bufs[b], acc.at[dstb2.at[0]], add=True)

            @pl.when(j < (NCH - 1) // 2 - 1)
            def _start_next():
                unpack_src(cidx + 2, b)
                pltpu.async_copy(y_hbm.at[srcb2.at[b]], bufs[b], gsems[b])

        return carry

    lax.fori_loop(0, (NCH - 1) // 2, body, 0)
    unpack_src(NCH - 1, 0)
    pltpu.async_copy(y_hbm.at[srcb2.at[0]], bufs[0], gsems[0]).wait()
    unpack_dst(NCH - 1)
    pltpu.sync_copy(bufs[0], acc.at[dstb2.at[0]], add=True)

    unpack(NCH * CHUNK, TAIL, srcbt, False)
    pltpu.async_copy(y_hbm.at[srcbt.at[0]], bufs[1].at[pl.ds(0, TAIL)], gsems[1]).wait()
    unpack(NCH * CHUNK, TAIL, dstbt, True)
    pltpu.sync_copy(bufs[1].at[pl.ds(0, TAIL)], acc.at[dstbt.at[0]], add=True)

    plsc.subcore_barrier()
    pltpu.sync_copy(acc.at[pl.ds(s * RPS, RPS)], out_hbm.at[c, pl.ds(s * RPS, RPS)])


BLK = 2000


def _dis_from(dt):
    deg = dt[:, 0:1] + dt[:, 1:2] + 1.0
    return lax.rsqrt(deg)


def _tc_first_body(x_ref, w_ref, dt_ref, y_ref):
    dis = _dis_from(dt_ref[...])
    xw = jnp.dot(x_ref[...], w_ref[...], preferred_element_type=jnp.float32)
    y_ref[...] = dis * xw


def _tc_mid_body(agg_ref, y1_ref, dt_ref, b_ref, w_ref, y2_ref):
    dis = _dis_from(dt_ref[...])
    a = agg_ref[...]
    h = jnp.maximum(dis * (a[0] + a[1] + y1_ref[...]) + b_ref[...], 0.0)
    y2_ref[...] = dis * jnp.dot(h, w_ref[...], preferred_element_type=jnp.float32)


def _tc_last_body(agg_ref, y2_ref, dt_ref, b_ref, out_ref):
    dis = _dis_from(dt_ref[...])
    a = agg_ref[...]
    out_ref[...] = dis * (a[0] + a[1] + y2_ref[...]) + b_ref[...]


def _row_spec(width):
    return pl.BlockSpec((BLK, width), lambda i: (i, 0))


def _pair_spec(width):
    return pl.BlockSpec((2, BLK, width), lambda i: (0, i, 0))


def _full_spec(shape):
    return pl.BlockSpec(shape, lambda i: tuple(0 for _ in shape))


def _tc_first(x, w, dt):
    return pl.pallas_call(
        _tc_first_body,
        grid=(N_NODES // BLK,),
        in_specs=[_row_spec(D), _full_spec((D, D)), _row_spec(2)],
        out_specs=_row_spec(D),
        out_shape=jax.ShapeDtypeStruct((N_NODES, D), jnp.float32),
    )(x, w, dt)


def _tc_mid(agg, y1, dt, b, w):
    return pl.pallas_call(
        _tc_mid_body,
        grid=(N_NODES // BLK,),
        in_specs=[
            _pair_spec(D),
            _row_spec(D),
            _row_spec(2),
            _full_spec((1, D)),
            _full_spec((D, D)),
        ],
        out_specs=_row_spec(D),
        out_shape=jax.ShapeDtypeStruct((N_NODES, D), jnp.float32),
    )(agg, y1, dt, b, w)


def _tc_last(agg, y2, dt, b):
    return pl.pallas_call(
        _tc_last_body,
        grid=(N_NODES // BLK,),
        in_specs=[_pair_spec(D), _row_spec(D), _row_spec(2), _full_spec((1, D))],
        out_specs=_row_spec(D),
        out_shape=jax.ShapeDtypeStruct((N_NODES, D), jnp.float32),
    )(agg, y2, dt, b)


def kernel(x, edge_index, W1, b1, W2, b2):
    src = edge_index[0].astype(jnp.int32)
    dst = edge_index[1].astype(jnp.int32)
    dst_flat = dst.reshape(NW, EPT)
    packed = (src | (dst << 16)).reshape(NW, EPT)
    b1r = b1.reshape(1, D)
    b2r = b2.reshape(1, D)

    dp = _sc_deg(dst_flat)
    dt = dp.T
    y1 = _tc_first(x, W1, dt)
    agg1 = _sc_agg(y1, packed)
    y2 = _tc_mid(agg1, y1, dt, b1r, W2)
    agg2 = _sc_agg(y2, packed)
    return _tc_last(agg2, y2, dt, b2r)
